# baseline (speedup 1.0000x reference)
import functools
import math

import jax
import jax.numpy as jnp
from jax import lax
from jax.experimental import pallas as pl
from jax.experimental.pallas import tpu as pltpu
from jax.experimental.pallas import tpu_sc as plsc

_KS = 3
_NS = [800, 1600, 3200, 6400, 12800, 25600]
_LVLS = ["l0", "l1", "l2", "l3", "l4", "l5"]
_NW = 32
_F32 = jnp.float32
_I32 = jnp.int32


def _mesh():
    return plsc.VectorSubcoreMesh(core_axis_name="c", subcore_axis_name="s")


_SC_PARAMS = pltpu.CompilerParams(
    needs_layout_passes=False, use_tc_tiling_on_sc=False)


def _wid():
    return lax.axis_index("s") * 2 + lax.axis_index("c")


def _round_up(x, m):
    return (x + m - 1) // m * m


@functools.lru_cache(maxsize=None)
def _make_deg(n):
    e = n * 16
    npad = _round_up(n, 256)
    epw = e // _NW
    ce = min(epw, 1600)
    nch = epw // ce

    @functools.partial(
        pl.kernel,
        out_type=jax.ShapeDtypeStruct((_NW, npad), _F32),
        mesh=_mesh(),
        compiler_params=_SC_PARAMS,
        scratch_types=[
            pltpu.VMEM((npad,), _F32),
            pltpu.VMEM((2 * ce,), _I32),
            pltpu.VMEM((2 * ce,), _I32),
            pltpu.SemaphoreType.DMA,
            pltpu.SemaphoreType.DMA,
        ],
    )
    def deg_kernel(ew_hbm, out_hbm, part_v, e0_v, e1_v, sem0, sem1):
        w = _wid()
        zero16 = jnp.zeros((16,), _F32)

        @plsc.parallel_loop(0, npad // 16, unroll=8)
        def _(i):
            part_v[pl.ds(i * 16, 16)] = zero16
        ebase = 2 * w * epw

        def compute(buf):
            @plsc.parallel_loop(0, ce // 16, unroll=8)
            def _(i):
                pk16 = buf[pl.ds(i * 32, 16)]
                d16 = lax.shift_right_logical(pk16, 16)
                w16 = plsc.bitcast(buf[pl.ds(i * 32 + 16, 16)], _F32)
                plsc.addupdate_scatter(part_v, [d16], w16)

        if nch == 1:
            pltpu.sync_copy(ew_hbm.at[pl.ds(ebase, 2 * ce)], e0_v)
            compute(e0_v)
        else:
            pltpu.async_copy(ew_hbm.at[pl.ds(ebase, 2 * ce)], e0_v, sem0)

            def pair(p, _):
                c0 = 2 * p
                pltpu.async_copy(
                    ew_hbm.at[pl.ds(ebase + (c0 + 1) * 2 * ce, 2 * ce)],
                    e1_v, sem1)
                pltpu.make_async_copy(
                    ew_hbm.at[pl.ds(ebase + c0 * 2 * ce, 2 * ce)],
                    e0_v, sem0).wait()
                compute(e0_v)

                @pl.when(c0 + 2 < nch)
                def _():
                    pltpu.async_copy(
                        ew_hbm.at[pl.ds(ebase + (c0 + 2) * 2 * ce, 2 * ce)],
                        e0_v, sem0)

                pltpu.make_async_copy(
                    ew_hbm.at[pl.ds(ebase + (c0 + 1) * 2 * ce, 2 * ce)],
                    e1_v, sem1).wait()
                compute(e1_v)
                return 0

            lax.fori_loop(0, nch // 2, pair, 0)
        pltpu.sync_copy(part_v, out_hbm.at[w])

    return deg_kernel


@functools.lru_cache(maxsize=None)
def _make_isd(npad):
    def body(parts_ref, out_ref):
        deg = jnp.sum(parts_ref[...], axis=0, keepdims=True) + 1e-6
        out_ref[...] = lax.rsqrt(deg)

    return pl.pallas_call(
        body,
        out_shape=jax.ShapeDtypeStruct((1, npad), _F32),
    )


@functools.lru_cache(maxsize=None)
def _make_wn(n):
    e = n * 16
    npad = _round_up(n, 256)
    epw = e // _NW
    ce = min(epw, 1600)
    nch = epw // ce

    @functools.partial(
        pl.kernel,
        out_type=jax.ShapeDtypeStruct((e,), _F32),
        mesh=_mesh(),
        compiler_params=_SC_PARAMS,
        scratch_types=[
            pltpu.VMEM((npad,), _F32),
            pltpu.VMEM((2 * ce,), _I32),
            pltpu.VMEM((2 * ce,), _I32),
            pltpu.VMEM((ce,), _F32),
            pltpu.SemaphoreType.DMA,
            pltpu.SemaphoreType.DMA,
        ],
    )
    def wn_kernel(ew_hbm, isd_hbm, out_hbm, isd_v, e0_v, e1_v, o_v,
                  sem0, sem1):
        w = _wid()
        pltpu.sync_copy(isd_hbm.at[0], isd_v)
        ebase = w * epw

        def compute(buf, ch):
            @plsc.parallel_loop(0, ce // 16, unroll=8)
            def _(i):
                pk16 = buf[pl.ds(i * 32, 16)]
                s16 = jnp.bitwise_and(pk16, 0xFFFF)
                d16 = lax.shift_right_logical(pk16, 16)
                w16 = plsc.bitcast(buf[pl.ds(i * 32 + 16, 16)], _F32)
                a = plsc.load_gather(isd_v, [s16])
                b = plsc.load_gather(isd_v, [d16])
                o_v[pl.ds(i * 16, 16)] = w16 * a * b
            pltpu.sync_copy(o_v, out_hbm.at[pl.ds(ebase + ch * ce, ce)])

        if nch == 1:
            pltpu.sync_copy(ew_hbm.at[pl.ds(2 * ebase, 2 * ce)], e0_v)
            compute(e0_v, 0)
        else:
            pltpu.async_copy(ew_hbm.at[pl.ds(2 * ebase, 2 * ce)], e0_v, sem0)

            def pair(p, _):
                c0 = 2 * p
                pltpu.async_copy(
                    ew_hbm.at[pl.ds(2 * (ebase + (c0 + 1) * ce), 2 * ce)],
                    e1_v, sem1)
                pltpu.make_async_copy(
                    ew_hbm.at[pl.ds(2 * (ebase + c0 * ce), 2 * ce)],
                    e0_v, sem0).wait()
                compute(e0_v, c0)

                @pl.when(c0 + 2 < nch)
                def _():
                    pltpu.async_copy(
                        ew_hbm.at[pl.ds(2 * (ebase + (c0 + 2) * ce), 2 * ce)],
                        e0_v, sem0)

                pltpu.make_async_copy(
                    ew_hbm.at[pl.ds(2 * (ebase + (c0 + 1) * ce), 2 * ce)],
                    e1_v, sem1).wait()
                compute(e1_v, c0 + 1)
                return 0

            lax.fori_loop(0, nch // 2, pair, 0)

    return wn_kernel


@functools.lru_cache(maxsize=None)
def _make_apply(n, c_cols, cheb):
    e = n * 16
    ce = 3200
    nch = e // ce
    half = nch // 2
    budget = 112000
    ncol = max(1, min(budget // (2 * n), 32, -(-c_cols // _NW)))
    nsweep = -(-c_cols // (_NW * ncol))

    scratch = [
        pltpu.VMEM((ncol * n,), _F32),
        pltpu.VMEM((ncol * n,), _F32),
        pltpu.VMEM((2 * ce,), _I32),
        pltpu.VMEM((2 * ce,), _I32),
        pltpu.SemaphoreType.DMA,
        pltpu.SemaphoreType.DMA,
    ]

    def body(x_hbm, ew_hbm, *rest):
        if cheb:
            prev_hbm, out_hbm, x_v, a_v, e0_v, e1_v, sem0, sem1 = rest
        else:
            out_hbm, x_v, a_v, e0_v, e1_v, sem0, sem1 = rest
        w = _wid()

        unroll = max(1, min(16, 64 // max(ncol, 1)))

        def compute(buf):
            @plsc.parallel_loop(0, ce // 16, unroll=unroll)
            def _(i):
                pk16 = buf[pl.ds(i * 32, 16)]
                s16 = jnp.bitwise_and(pk16, 0xFFFF)
                d16 = lax.shift_right_logical(pk16, 16)
                w16 = plsc.bitcast(buf[pl.ds(i * 32 + 16, 16)], _F32)
                for j in range(ncol):
                    v = plsc.load_gather(x_v, [s16 + j * n])
                    plsc.addupdate_scatter(a_v, [d16 + j * n], v * w16)

        for sw in range(nsweep):
            base = (sw * _NW + w) * ncol

            for j in range(ncol):
                col = base + j

                @pl.when(col < c_cols)
                def _():
                    pltpu.sync_copy(x_hbm.at[col], x_v.at[pl.ds(j * n, n)])
                    pltpu.sync_copy(x_hbm.at[col], a_v.at[pl.ds(j * n, n)])

            @pl.when(base < c_cols)
            def _():
                pltpu.async_copy(ew_hbm.at[pl.ds(0, 2 * ce)], e0_v, sem0)

                def pair(p, _):
                    c0 = 2 * p
                    pltpu.async_copy(
                        ew_hbm.at[pl.ds((c0 + 1) * 2 * ce, 2 * ce)], e1_v, sem1)
                    pltpu.make_async_copy(
                        ew_hbm.at[pl.ds(c0 * 2 * ce, 2 * ce)], e0_v, sem0).wait()
                    compute(e0_v)

                    @pl.when(c0 + 2 < nch)
                    def _():
                        pltpu.async_copy(
                            ew_hbm.at[pl.ds((c0 + 2) * 2 * ce, 2 * ce)],
                            e0_v, sem0)

                    pltpu.make_async_copy(
                        ew_hbm.at[pl.ds((c0 + 1) * 2 * ce, 2 * ce)],
                        e1_v, sem1).wait()
                    compute(e1_v)
                    return 0

                lax.fori_loop(0, half, pair, 0)

            for j in range(ncol):
                col = base + j

                @pl.when(col < c_cols)
                def _():
                    if not cheb:
                        pltpu.sync_copy(a_v.at[pl.ds(j * n, n)], out_hbm.at[col])
                    else:
                        pltpu.sync_copy(prev_hbm.at[col], x_v.at[pl.ds(j * n, n)])

                        jj = j * n

                        @plsc.parallel_loop(0, n // 16, unroll=8)
                        def _(i):
                            av = a_v[pl.ds(jj + i * 16, 16)]
                            pv = x_v[pl.ds(jj + i * 16, 16)]
                            x_v[pl.ds(jj + i * 16, 16)] = 2.0 * av - pv
                        pltpu.sync_copy(x_v.at[pl.ds(j * n, n)], out_hbm.at[col])

    return functools.partial(
        pl.kernel,
        out_type=jax.ShapeDtypeStruct((c_cols, n), _F32),
        mesh=_mesh(),
        compiler_params=_SC_PARAMS,
        scratch_types=scratch,
    )(body)


@functools.lru_cache(maxsize=None)
def _make_dec_apply(n, c_cols):
    e = n * 16
    ce = 3200
    nch = e // ce
    half = nch // 2
    budget = 112000
    ncol = max(1, min(budget // (2 * n), 32, -(-c_cols // _NW)))
    nsweep = -(-c_cols // (_NW * ncol))

    scratch = [
        pltpu.VMEM((ncol * n,), _F32),
        pltpu.VMEM((ncol * n,), _F32),
        pltpu.VMEM((2 * ce,), _I32),
        pltpu.VMEM((2 * ce,), _I32),
        pltpu.SemaphoreType.DMA,
        pltpu.SemaphoreType.DMA,
    ]

    @functools.partial(
        pl.kernel,
        out_type=jax.ShapeDtypeStruct((c_cols, n), _F32),
        mesh=_mesh(),
        compiler_params=_SC_PARAMS,
        scratch_types=scratch,
    )
    def dec_kernel(z2_hbm, z1_hbm, base_hbm, ew_hbm, out_hbm,
                   x_v, a_v, e0_v, e1_v, sem0, sem1):
        w = _wid()
        unroll = max(1, min(16, 64 // max(ncol, 1)))

        def compute(buf):
            @plsc.parallel_loop(0, ce // 16, unroll=unroll)
            def _(i):
                pk16 = buf[pl.ds(i * 32, 16)]
                s16 = jnp.bitwise_and(pk16, 0xFFFF)
                d16 = lax.shift_right_logical(pk16, 16)
                w16 = plsc.bitcast(buf[pl.ds(i * 32 + 16, 16)], _F32)
                for j in range(ncol):
                    v = plsc.load_gather(x_v, [s16 + j * n])
                    plsc.addupdate_scatter(a_v, [d16 + j * n], v * w16)

        def edge_pass():
            pltpu.async_copy(ew_hbm.at[pl.ds(0, 2 * ce)], e0_v, sem0)

            def pair(p, _):
                c0 = 2 * p
                pltpu.async_copy(
                    ew_hbm.at[pl.ds((c0 + 1) * 2 * ce, 2 * ce)], e1_v, sem1)
                pltpu.make_async_copy(
                    ew_hbm.at[pl.ds(c0 * 2 * ce, 2 * ce)], e0_v, sem0).wait()
                compute(e0_v)

                @pl.when(c0 + 2 < nch)
                def _():
                    pltpu.async_copy(
                        ew_hbm.at[pl.ds((c0 + 2) * 2 * ce, 2 * ce)],
                        e0_v, sem0)

                pltpu.make_async_copy(
                    ew_hbm.at[pl.ds((c0 + 1) * 2 * ce, 2 * ce)],
                    e1_v, sem1).wait()
                compute(e1_v)
                return 0

            lax.fori_loop(0, half, pair, 0)

        for sw in range(nsweep):
            base = (sw * _NW + w) * ncol

            for j in range(ncol):
                col = base + j

                @pl.when(col < c_cols)
                def _():
                    pltpu.sync_copy(z2_hbm.at[col], x_v.at[pl.ds(j * n, n)])
                    pltpu.sync_copy(z2_hbm.at[col], a_v.at[pl.ds(j * n, n)])

            @pl.when(base < c_cols)
            def _():
                edge_pass()

            for j in range(ncol):
                col = base + j

                @pl.when(col < c_cols)
                def _():
                    pltpu.sync_copy(z1_hbm.at[col], x_v.at[pl.ds(j * n, n)])

            @pl.when(base < c_cols)
            def _():
                @plsc.parallel_loop(0, ncol * n // 16, unroll=8)
                def _(i):
                    t = x_v[pl.ds(i * 16, 16)] + 2.0 * a_v[pl.ds(i * 16, 16)]
                    x_v[pl.ds(i * 16, 16)] = t
                    a_v[pl.ds(i * 16, 16)] = t

                edge_pass()

            for j in range(ncol):
                col = base + j
                jj = j * n

                @pl.when(col < c_cols)
                def _():
                    pltpu.sync_copy(base_hbm.at[col], x_v.at[pl.ds(jj, n)])

                    @plsc.parallel_loop(0, n // 16, unroll=8)
                    def _(i):
                        t = x_v[pl.ds(jj + i * 16, 16)] + a_v[pl.ds(jj + i * 16, 16)]
                        x_v[pl.ds(jj + i * 16, 16)] = jnp.maximum(t, 0.0)

                    pltpu.sync_copy(x_v.at[pl.ds(jj, n)], out_hbm.at[col])

    return dec_kernel


@functools.lru_cache(maxsize=None)
def _make_mm(f, m, n, b, has_bias, f2, act):
    nb = min(1024, n)
    grid = (b, -(-n // nb))

    def body(*refs):
        idx = 0
        w_ref = refs[idx]; idx += 1
        t_ref = refs[idx]; idx += 1
        if has_bias:
            bias_ref = refs[idx]; idx += 1
        if f2:
            w2_ref = refs[idx]; idx += 1
            t2_ref = refs[idx]; idx += 1
        out_ref = refs[idx]
        y = lax.dot_general(
            w_ref[...], t_ref[0],
            (((0,), (0,)), ((), ())),
            precision=lax.Precision.HIGHEST,
            preferred_element_type=_F32,
        )
        if f2:
            y = y + lax.dot_general(
                w2_ref[...], t2_ref[0],
                (((0,), (0,)), ((), ())),
                precision=lax.Precision.HIGHEST,
                preferred_element_type=_F32,
            )
        if has_bias:
            y = y + bias_ref[...]
        if act == "relu":
            y = jnp.maximum(y, 0.0)
        elif act == "lsm":
            y = y - jnp.max(y, axis=0, keepdims=True)
            y = y - jnp.log(jnp.sum(jnp.exp(y), axis=0, keepdims=True))
        out_ref[0] = y

    in_specs = [
        pl.BlockSpec((f, m), lambda bi, ni: (0, 0)),
        pl.BlockSpec((1, f, nb), lambda bi, ni: (bi, 0, ni)),
    ]
    if has_bias:
        in_specs.append(pl.BlockSpec((m, 1), lambda bi, ni: (0, 0)))
    if f2:
        in_specs.append(pl.BlockSpec((f2, m), lambda bi, ni: (0, 0)))
        in_specs.append(pl.BlockSpec((1, f2, nb), lambda bi, ni: (bi, 0, ni)))

    return pl.pallas_call(
        body,
        grid=grid,
        in_specs=in_specs,
        out_specs=pl.BlockSpec((1, m, nb), lambda bi, ni: (bi, 0, ni)),
        out_shape=jax.ShapeDtypeStruct((b, m, n), _F32),
    )


@functools.lru_cache(maxsize=None)
def _make_max(r, ncols):
    br = min(r, 256)
    bn = min(ncols, 2048)
    grid = (-(-r // br), -(-ncols // bn))

    def body(a_ref, b_ref, o_ref):
        o_ref[...] = jnp.maximum(a_ref[...], b_ref[...])

    spec = pl.BlockSpec((br, bn), lambda i, j: (i, j))
    return pl.pallas_call(
        body,
        grid=grid,
        in_specs=[spec, spec],
        out_specs=spec,
        out_shape=jax.ShapeDtypeStruct((r, ncols), _F32),
    )


def _cheb_T(xbc, graph):
    ew, n = graph
    b, cin, _ = xbc.shape
    c = b * cin
    x2 = xbc.reshape(c, n)
    t1 = _make_apply(n, c, False)(x2, ew)
    t2 = _make_apply(n, c, True)(t1, ew, x2)
    return jnp.concatenate(
        [xbc, t1.reshape(b, cin, n), t2.reshape(b, cin, n)], axis=1)


def _conv_k3(xbc, p, graph, act):
    t = _cheb_T(xbc, graph)
    b, f, n = t.shape
    m = p["W"].shape[2]
    wf = p["W"].reshape(f, m)
    bias = p["b"].reshape(m, 1)
    return _make_mm(f, m, n, b, True, 0, act)(wf, t, bias)


def _conv_k3_commuted(xbc, p, graph):
    ew, n = graph
    b, cin, _ = xbc.shape
    cout = p["W"].shape[2]
    w0, w1, w2 = p["W"][0], p["W"][1], p["W"][2]
    wp = jnp.concatenate([w0 - w2, w1, w2], axis=1)
    bias3 = jnp.concatenate(
        [p["b"], jnp.zeros((2 * cout,), _F32)]).reshape(3 * cout, 1)
    y = _make_mm(cin, 3 * cout, n, b, True, 0, "none")(wp, xbc, bias3)
    c = b * cout
    base = y[:, :cout].reshape(c, n)
    z1 = y[:, cout:2 * cout].reshape(c, n)
    z2 = y[:, 2 * cout:].reshape(c, n)
    out = _make_dec_apply(n, c)(z2, z1, base, ew)
    return out.reshape(b, cout, n)


def _res_block(xbc, p, graph):
    if p["conv1"]["W"].shape[2] < xbc.shape[1]:
        h = _conv_k3_commuted(xbc, p["conv1"], graph)
    else:
        h = _conv_k3(xbc, p["conv1"], graph, "relu")
    t = _cheb_T(h, graph)
    b, f, n = t.shape
    cin = xbc.shape[1]
    m = p["conv2"]["W"].shape[2]
    w2f = p["conv2"]["W"].reshape(f, m)
    bias = p["conv2"]["b"].reshape(m, 1)
    wscf = p["sc"]["W"].reshape(cin, m)
    return _make_mm(f, m, n, b, True, cin, "relu")(w2f, t, bias, wscf, xbc)


def _pool(t):
    b, c, n = t.shape
    a = t[:, :, 0::2].reshape(b * c, n // 2)
    bb = t[:, :, 1::2].reshape(b * c, n // 2)
    return _make_max(b * c, n // 2)(a, bb).reshape(b, c, n // 2)


def _unpool(t):
    return jnp.repeat(t, 2, axis=2)


def kernel(x, params, edge_src, edge_dst, edge_w):
    graphs = {}
    for i, lvl in enumerate(_LVLS):
        n = _NS[i]
        src = edge_src[lvl].astype(_I32)
        dst = edge_dst[lvl].astype(_I32)
        pk = jnp.bitwise_or(src, dst << 16)
        ew = edge_w[lvl].astype(_F32)
        ew_i = lax.bitcast_convert_type(ew, _I32)
        ewp = jnp.stack(
            [pk.reshape(-1, 16), ew_i.reshape(-1, 16)], axis=1).reshape(-1)
        parts = _make_deg(n)(ewp)
        isd = _make_isd(_round_up(n, 256))(parts)
        wn = _make_wn(n)(ewp, isd)
        wn_i = lax.bitcast_convert_type(-wn, _I32)
        epk = jnp.stack(
            [pk.reshape(-1, 16), wn_i.reshape(-1, 16)], axis=1).reshape(-1)
        graphs[lvl] = (epk, n)

    h = _conv_k3(x, params["enc_conv"], graphs["l5"], "relu")
    e5 = _res_block(h, params["enc_b5"], graphs["l5"])
    e4 = _res_block(_pool(e5), params["enc_b4"], graphs["l4"])
    e3 = _res_block(_pool(e4), params["enc_b3"], graphs["l3"])
    e2 = _res_block(_pool(e3), params["enc_b2"], graphs["l2"])
    e1 = _res_block(_pool(e2), params["enc_b1"], graphs["l1"])
    e0 = _res_block(_pool(e1), params["enc_b0"], graphs["l0"])
    d1 = _res_block(jnp.concatenate([_unpool(e0), e1], axis=1),
                    params["dec_b1"], graphs["l1"])
    d2 = _res_block(jnp.concatenate([_unpool(d1), e2], axis=1),
                    params["dec_b2"], graphs["l2"])
    d3 = _res_block(jnp.concatenate([_unpool(d2), e3], axis=1),
                    params["dec_b3"], graphs["l3"])
    d4 = _res_block(jnp.concatenate([_unpool(d3), e4], axis=1),
                    params["dec_b4"], graphs["l4"])
    d5 = _res_block(jnp.concatenate([_unpool(d4), e5], axis=1),
                    params["dec_b5"], graphs["l5"])

    b, cin, n = d5.shape
    wdec = params["dec_conv"]["W"].reshape(cin, 10)
    return _make_mm(cin, 10, n, b, False, 0, "lsm")(wdec, d5)


# speedup vs baseline: 1.0254x; 1.0254x over previous
import functools
import math

import jax
import jax.numpy as jnp
from jax import lax
from jax.experimental import pallas as pl
from jax.experimental.pallas import tpu as pltpu
from jax.experimental.pallas import tpu_sc as plsc

_KS = 3
_NS = [800, 1600, 3200, 6400, 12800, 25600]
_LVLS = ["l0", "l1", "l2", "l3", "l4", "l5"]
_NW = 32
_F32 = jnp.float32
_I32 = jnp.int32


def _mesh():
    return plsc.VectorSubcoreMesh(core_axis_name="c", subcore_axis_name="s")


_SC_PARAMS = pltpu.CompilerParams(
    needs_layout_passes=False, use_tc_tiling_on_sc=False)


def _wid():
    return lax.axis_index("s") * 2 + lax.axis_index("c")


def _round_up(x, m):
    return (x + m - 1) // m * m


@functools.lru_cache(maxsize=None)
def _make_deg(n):
    e = n * 16
    npad = _round_up(n, 256)
    epw = e // _NW
    ce = min(epw, 1600)
    nch = epw // ce

    @functools.partial(
        pl.kernel,
        out_type=jax.ShapeDtypeStruct((_NW, npad), _F32),
        mesh=_mesh(),
        compiler_params=_SC_PARAMS,
        scratch_types=[
            pltpu.VMEM((npad,), _F32),
            pltpu.VMEM((2 * ce,), _I32),
            pltpu.VMEM((2 * ce,), _I32),
            pltpu.SemaphoreType.DMA,
            pltpu.SemaphoreType.DMA,
        ],
    )
    def deg_kernel(ew_hbm, out_hbm, part_v, e0_v, e1_v, sem0, sem1):
        w = _wid()
        zero16 = jnp.zeros((16,), _F32)

        @plsc.parallel_loop(0, npad // 16, unroll=8)
        def _(i):
            part_v[pl.ds(i * 16, 16)] = zero16
        ebase = 2 * w * epw

        def compute(buf):
            @plsc.parallel_loop(0, ce // 16, unroll=8)
            def _(i):
                pk16 = buf[pl.ds(i * 32, 16)]
                d16 = lax.shift_right_logical(pk16, 16)
                w16 = plsc.bitcast(buf[pl.ds(i * 32 + 16, 16)], _F32)
                plsc.addupdate_scatter(part_v, [d16], w16)

        if nch == 1:
            pltpu.sync_copy(ew_hbm.at[pl.ds(ebase, 2 * ce)], e0_v)
            compute(e0_v)
        else:
            pltpu.async_copy(ew_hbm.at[pl.ds(ebase, 2 * ce)], e0_v, sem0)

            def pair(p, _):
                c0 = 2 * p
                pltpu.async_copy(
                    ew_hbm.at[pl.ds(ebase + (c0 + 1) * 2 * ce, 2 * ce)],
                    e1_v, sem1)
                pltpu.make_async_copy(
                    ew_hbm.at[pl.ds(ebase + c0 * 2 * ce, 2 * ce)],
                    e0_v, sem0).wait()
                compute(e0_v)

                @pl.when(c0 + 2 < nch)
                def _():
                    pltpu.async_copy(
                        ew_hbm.at[pl.ds(ebase + (c0 + 2) * 2 * ce, 2 * ce)],
                        e0_v, sem0)

                pltpu.make_async_copy(
                    ew_hbm.at[pl.ds(ebase + (c0 + 1) * 2 * ce, 2 * ce)],
                    e1_v, sem1).wait()
                compute(e1_v)
                return 0

            lax.fori_loop(0, nch // 2, pair, 0)
        pltpu.sync_copy(part_v, out_hbm.at[w])

    return deg_kernel


@functools.lru_cache(maxsize=None)
def _make_isd(npad):
    def body(parts_ref, out_ref):
        deg = jnp.sum(parts_ref[...], axis=0, keepdims=True) + 1e-6
        out_ref[...] = lax.rsqrt(deg)

    return pl.pallas_call(
        body,
        out_shape=jax.ShapeDtypeStruct((1, npad), _F32),
    )


@functools.lru_cache(maxsize=None)
def _make_wn(n):
    e = n * 16
    npad = _round_up(n, 256)
    epw = e // _NW
    ce = min(epw, 1600)
    nch = epw // ce

    @functools.partial(
        pl.kernel,
        out_type=jax.ShapeDtypeStruct((e,), _F32),
        mesh=_mesh(),
        compiler_params=_SC_PARAMS,
        scratch_types=[
            pltpu.VMEM((npad,), _F32),
            pltpu.VMEM((2 * ce,), _I32),
            pltpu.VMEM((2 * ce,), _I32),
            pltpu.VMEM((ce,), _F32),
            pltpu.SemaphoreType.DMA,
            pltpu.SemaphoreType.DMA,
        ],
    )
    def wn_kernel(ew_hbm, isd_hbm, out_hbm, isd_v, e0_v, e1_v, o_v,
                  sem0, sem1):
        w = _wid()
        pltpu.sync_copy(isd_hbm.at[0], isd_v)
        ebase = w * epw

        def compute(buf, ch):
            @plsc.parallel_loop(0, ce // 16, unroll=8)
            def _(i):
                pk16 = buf[pl.ds(i * 32, 16)]
                s16 = jnp.bitwise_and(pk16, 0xFFFF)
                d16 = lax.shift_right_logical(pk16, 16)
                w16 = plsc.bitcast(buf[pl.ds(i * 32 + 16, 16)], _F32)
                a = plsc.load_gather(isd_v, [s16])
                b = plsc.load_gather(isd_v, [d16])
                o_v[pl.ds(i * 16, 16)] = w16 * a * b
            pltpu.sync_copy(o_v, out_hbm.at[pl.ds(ebase + ch * ce, ce)])

        if nch == 1:
            pltpu.sync_copy(ew_hbm.at[pl.ds(2 * ebase, 2 * ce)], e0_v)
            compute(e0_v, 0)
        else:
            pltpu.async_copy(ew_hbm.at[pl.ds(2 * ebase, 2 * ce)], e0_v, sem0)

            def pair(p, _):
                c0 = 2 * p
                pltpu.async_copy(
                    ew_hbm.at[pl.ds(2 * (ebase + (c0 + 1) * ce), 2 * ce)],
                    e1_v, sem1)
                pltpu.make_async_copy(
                    ew_hbm.at[pl.ds(2 * (ebase + c0 * ce), 2 * ce)],
                    e0_v, sem0).wait()
                compute(e0_v, c0)

                @pl.when(c0 + 2 < nch)
                def _():
                    pltpu.async_copy(
                        ew_hbm.at[pl.ds(2 * (ebase + (c0 + 2) * ce), 2 * ce)],
                        e0_v, sem0)

                pltpu.make_async_copy(
                    ew_hbm.at[pl.ds(2 * (ebase + (c0 + 1) * ce), 2 * ce)],
                    e1_v, sem1).wait()
                compute(e1_v, c0 + 1)
                return 0

            lax.fori_loop(0, nch // 2, pair, 0)

    return wn_kernel


@functools.lru_cache(maxsize=None)
def _make_apply(n, c_cols, cheb):
    e = n * 16
    ce = 3200
    nch = e // ce
    half = nch // 2
    budget = 112000
    ncol = max(1, min(budget // (2 * n), 32, -(-c_cols // _NW)))
    nsweep = -(-c_cols // (_NW * ncol))

    scratch = [
        pltpu.VMEM((ncol * n,), _F32),
        pltpu.VMEM((ncol * n,), _F32),
        pltpu.VMEM((2 * ce,), _I32),
        pltpu.VMEM((2 * ce,), _I32),
        pltpu.SemaphoreType.DMA,
        pltpu.SemaphoreType.DMA,
    ]

    def body(x_hbm, ew_hbm, *rest):
        if cheb:
            prev_hbm, out_hbm, x_v, a_v, e0_v, e1_v, sem0, sem1 = rest
        else:
            out_hbm, x_v, a_v, e0_v, e1_v, sem0, sem1 = rest
        w = _wid()

        unroll = max(1, min(16, 64 // max(ncol, 1)))

        def compute(buf):
            @plsc.parallel_loop(0, ce // 16, unroll=unroll)
            def _(i):
                pk16 = buf[pl.ds(i * 32, 16)]
                s16 = jnp.bitwise_and(pk16, 0xFFFF)
                d16 = lax.shift_right_logical(pk16, 16)
                w16 = plsc.bitcast(buf[pl.ds(i * 32 + 16, 16)], _F32)
                for j in range(ncol):
                    v = plsc.load_gather(x_v, [s16 + j * n])
                    plsc.addupdate_scatter(a_v, [d16 + j * n], v * w16)

        for sw in range(nsweep):
            base = (sw * _NW + w) * ncol

            for j in range(ncol):
                col = base + j

                @pl.when(col < c_cols)
                def _():
                    pltpu.sync_copy(x_hbm.at[col], x_v.at[pl.ds(j * n, n)])
                    pltpu.sync_copy(x_hbm.at[col], a_v.at[pl.ds(j * n, n)])

            @pl.when(base < c_cols)
            def _():
                pltpu.async_copy(ew_hbm.at[pl.ds(0, 2 * ce)], e0_v, sem0)

                def pair(p, _):
                    c0 = 2 * p
                    pltpu.async_copy(
                        ew_hbm.at[pl.ds((c0 + 1) * 2 * ce, 2 * ce)], e1_v, sem1)
                    pltpu.make_async_copy(
                        ew_hbm.at[pl.ds(c0 * 2 * ce, 2 * ce)], e0_v, sem0).wait()
                    compute(e0_v)

                    @pl.when(c0 + 2 < nch)
                    def _():
                        pltpu.async_copy(
                            ew_hbm.at[pl.ds((c0 + 2) * 2 * ce, 2 * ce)],
                            e0_v, sem0)

                    pltpu.make_async_copy(
                        ew_hbm.at[pl.ds((c0 + 1) * 2 * ce, 2 * ce)],
                        e1_v, sem1).wait()
                    compute(e1_v)
                    return 0

                lax.fori_loop(0, half, pair, 0)

            for j in range(ncol):
                col = base + j

                @pl.when(col < c_cols)
                def _():
                    if not cheb:
                        pltpu.sync_copy(a_v.at[pl.ds(j * n, n)], out_hbm.at[col])
                    else:
                        pltpu.sync_copy(prev_hbm.at[col], x_v.at[pl.ds(j * n, n)])

                        jj = j * n

                        @plsc.parallel_loop(0, n // 16, unroll=8)
                        def _(i):
                            av = a_v[pl.ds(jj + i * 16, 16)]
                            pv = x_v[pl.ds(jj + i * 16, 16)]
                            x_v[pl.ds(jj + i * 16, 16)] = 2.0 * av - pv
                        pltpu.sync_copy(x_v.at[pl.ds(j * n, n)], out_hbm.at[col])

    return functools.partial(
        pl.kernel,
        out_type=jax.ShapeDtypeStruct((c_cols, n), _F32),
        mesh=_mesh(),
        compiler_params=_SC_PARAMS,
        scratch_types=scratch,
    )(body)


@functools.lru_cache(maxsize=None)
def _make_dec_apply(n, c_cols):
    e = n * 16
    ce = 3200
    nch = e // ce
    half = nch // 2
    budget = 112000
    ncol = max(1, min(budget // (2 * n), 32, -(-c_cols // _NW)))
    nsweep = -(-c_cols // (_NW * ncol))

    scratch = [
        pltpu.VMEM((ncol * n,), _F32),
        pltpu.VMEM((ncol * n,), _F32),
        pltpu.VMEM((2 * ce,), _I32),
        pltpu.VMEM((2 * ce,), _I32),
        pltpu.SemaphoreType.DMA,
        pltpu.SemaphoreType.DMA,
    ]

    @functools.partial(
        pl.kernel,
        out_type=jax.ShapeDtypeStruct((c_cols, n), _F32),
        mesh=_mesh(),
        compiler_params=_SC_PARAMS,
        scratch_types=scratch,
    )
    def dec_kernel(z2_hbm, z1_hbm, base_hbm, ew_hbm, out_hbm,
                   x_v, a_v, e0_v, e1_v, sem0, sem1):
        w = _wid()
        unroll = max(1, min(16, 64 // max(ncol, 1)))

        def compute(buf):
            @plsc.parallel_loop(0, ce // 16, unroll=unroll)
            def _(i):
                pk16 = buf[pl.ds(i * 32, 16)]
                s16 = jnp.bitwise_and(pk16, 0xFFFF)
                d16 = lax.shift_right_logical(pk16, 16)
                w16 = plsc.bitcast(buf[pl.ds(i * 32 + 16, 16)], _F32)
                for j in range(ncol):
                    v = plsc.load_gather(x_v, [s16 + j * n])
                    plsc.addupdate_scatter(a_v, [d16 + j * n], v * w16)

        def edge_pass():
            pltpu.async_copy(ew_hbm.at[pl.ds(0, 2 * ce)], e0_v, sem0)

            def pair(p, _):
                c0 = 2 * p
                pltpu.async_copy(
                    ew_hbm.at[pl.ds((c0 + 1) * 2 * ce, 2 * ce)], e1_v, sem1)
                pltpu.make_async_copy(
                    ew_hbm.at[pl.ds(c0 * 2 * ce, 2 * ce)], e0_v, sem0).wait()
                compute(e0_v)

                @pl.when(c0 + 2 < nch)
                def _():
                    pltpu.async_copy(
                        ew_hbm.at[pl.ds((c0 + 2) * 2 * ce, 2 * ce)],
                        e0_v, sem0)

                pltpu.make_async_copy(
                    ew_hbm.at[pl.ds((c0 + 1) * 2 * ce, 2 * ce)],
                    e1_v, sem1).wait()
                compute(e1_v)
                return 0

            lax.fori_loop(0, half, pair, 0)

        for sw in range(nsweep):
            base = (sw * _NW + w) * ncol

            for j in range(ncol):
                col = base + j

                @pl.when(col < c_cols)
                def _():
                    pltpu.sync_copy(z2_hbm.at[col], x_v.at[pl.ds(j * n, n)])
                    pltpu.sync_copy(z2_hbm.at[col], a_v.at[pl.ds(j * n, n)])

            @pl.when(base < c_cols)
            def _():
                edge_pass()

            for j in range(ncol):
                col = base + j

                @pl.when(col < c_cols)
                def _():
                    pltpu.sync_copy(z1_hbm.at[col], x_v.at[pl.ds(j * n, n)])

            @pl.when(base < c_cols)
            def _():
                @plsc.parallel_loop(0, ncol * n // 16, unroll=8)
                def _(i):
                    t = x_v[pl.ds(i * 16, 16)] + 2.0 * a_v[pl.ds(i * 16, 16)]
                    x_v[pl.ds(i * 16, 16)] = t
                    a_v[pl.ds(i * 16, 16)] = t

                edge_pass()

            for j in range(ncol):
                col = base + j
                jj = j * n

                @pl.when(col < c_cols)
                def _():
                    pltpu.sync_copy(base_hbm.at[col], x_v.at[pl.ds(jj, n)])

                    @plsc.parallel_loop(0, n // 16, unroll=8)
                    def _(i):
                        t = x_v[pl.ds(jj + i * 16, 16)] + a_v[pl.ds(jj + i * 16, 16)]
                        x_v[pl.ds(jj + i * 16, 16)] = jnp.maximum(t, 0.0)

                    pltpu.sync_copy(x_v.at[pl.ds(jj, n)], out_hbm.at[col])

    return dec_kernel


@functools.lru_cache(maxsize=None)
def _make_t1t2(n, c_cols):
    e = n * 16
    ce = 3200
    nch = e // ce
    half = nch // 2
    budget = 112000
    ncol = max(1, min(budget // (3 * n), 32, -(-c_cols // _NW)))
    nsweep = -(-c_cols // (_NW * ncol))

    scratch = [
        pltpu.VMEM((ncol * n,), _F32),
        pltpu.VMEM((ncol * n,), _F32),
        pltpu.VMEM((ncol * n,), _F32),
        pltpu.VMEM((2 * ce,), _I32),
        pltpu.VMEM((2 * ce,), _I32),
        pltpu.SemaphoreType.DMA,
        pltpu.SemaphoreType.DMA,
    ]

    @functools.partial(
        pl.kernel,
        out_type=[jax.ShapeDtypeStruct((c_cols, n), _F32),
                  jax.ShapeDtypeStruct((c_cols, n), _F32)],
        mesh=_mesh(),
        compiler_params=_SC_PARAMS,
        scratch_types=scratch,
    )
    def t1t2_kernel(x_hbm, ew_hbm, t1_hbm, t2_hbm,
                    x_v, t_v, a_v, e0_v, e1_v, sem0, sem1):
        w = _wid()
        unroll = max(1, min(16, 64 // max(ncol, 1)))

        def make_compute(src_v):
            def compute(buf):
                @plsc.parallel_loop(0, ce // 16, unroll=unroll)
                def _(i):
                    pk16 = buf[pl.ds(i * 32, 16)]
                    s16 = jnp.bitwise_and(pk16, 0xFFFF)
                    d16 = lax.shift_right_logical(pk16, 16)
                    w16 = plsc.bitcast(buf[pl.ds(i * 32 + 16, 16)], _F32)
                    for j in range(ncol):
                        v = plsc.load_gather(src_v, [s16 + j * n])
                        plsc.addupdate_scatter(a_v, [d16 + j * n], v * w16)
            return compute

        def edge_pass(compute):
            pltpu.async_copy(ew_hbm.at[pl.ds(0, 2 * ce)], e0_v, sem0)

            def pair(p, _):
                c0 = 2 * p
                pltpu.async_copy(
                    ew_hbm.at[pl.ds((c0 + 1) * 2 * ce, 2 * ce)], e1_v, sem1)
                pltpu.make_async_copy(
                    ew_hbm.at[pl.ds(c0 * 2 * ce, 2 * ce)], e0_v, sem0).wait()
                compute(e0_v)

                @pl.when(c0 + 2 < nch)
                def _():
                    pltpu.async_copy(
                        ew_hbm.at[pl.ds((c0 + 2) * 2 * ce, 2 * ce)],
                        e0_v, sem0)

                pltpu.make_async_copy(
                    ew_hbm.at[pl.ds((c0 + 1) * 2 * ce, 2 * ce)],
                    e1_v, sem1).wait()
                compute(e1_v)
                return 0

            lax.fori_loop(0, half, pair, 0)

        for sw in range(nsweep):
            base = (sw * _NW + w) * ncol

            for j in range(ncol):
                col = base + j

                @pl.when(col < c_cols)
                def _():
                    pltpu.sync_copy(x_hbm.at[col], x_v.at[pl.ds(j * n, n)])
                    pltpu.sync_copy(x_hbm.at[col], a_v.at[pl.ds(j * n, n)])

            @pl.when(base < c_cols)
            def _():
                edge_pass(make_compute(x_v))

            for j in range(ncol):
                col = base + j

                @pl.when(col < c_cols)
                def _():
                    pltpu.sync_copy(a_v.at[pl.ds(j * n, n)], t1_hbm.at[col])

            @pl.when(base < c_cols)
            def _():
                @plsc.parallel_loop(0, ncol * n // 16, unroll=8)
                def _(i):
                    t2x = 2.0 * a_v[pl.ds(i * 16, 16)]
                    t_v[pl.ds(i * 16, 16)] = t2x
                    a_v[pl.ds(i * 16, 16)] = t2x

                edge_pass(make_compute(t_v))

            for j in range(ncol):
                col = base + j
                jj = j * n

                @pl.when(col < c_cols)
                def _():
                    @plsc.parallel_loop(0, n // 16, unroll=8)
                    def _(i):
                        t = a_v[pl.ds(jj + i * 16, 16)] - x_v[pl.ds(jj + i * 16, 16)]
                        x_v[pl.ds(jj + i * 16, 16)] = t

                    pltpu.sync_copy(x_v.at[pl.ds(jj, n)], t2_hbm.at[col])

    return t1t2_kernel

@functools.lru_cache(maxsize=None)
def _make_mm(f, m, n, b, has_bias, f2, act, m2=None):
    m2 = m if m2 is None else m2
    nb = min(1024, n)
    grid = (b, -(-n // nb))

    def body(*refs):
        idx = 0
        w_ref = refs[idx]; idx += 1
        t_ref = refs[idx]; idx += 1
        if has_bias:
            bias_ref = refs[idx]; idx += 1
        if f2:
            w2_ref = refs[idx]; idx += 1
            t2_ref = refs[idx]; idx += 1
        out_ref = refs[idx]
        y = lax.dot_general(
            w_ref[...], t_ref[0],
            (((0,), (0,)), ((), ())),
            precision=lax.Precision.HIGHEST,
            preferred_element_type=_F32,
        )
        if f2:
            y2 = lax.dot_general(
                w2_ref[...], t2_ref[0],
                (((0,), (0,)), ((), ())),
                precision=lax.Precision.HIGHEST,
                preferred_element_type=_F32,
            )
            if m2 == m:
                y = y + y2
            else:
                y = jnp.concatenate([y[:m2] + y2, y[m2:]], axis=0)
        if has_bias:
            y = y + bias_ref[...]
        if act == "relu":
            y = jnp.maximum(y, 0.0)
        elif act == "lsm":
            y = y - jnp.max(y, axis=0, keepdims=True)
            y = y - jnp.log(jnp.sum(jnp.exp(y), axis=0, keepdims=True))
        out_ref[0] = y

    in_specs = [
        pl.BlockSpec((f, m), lambda bi, ni: (0, 0)),
        pl.BlockSpec((1, f, nb), lambda bi, ni: (bi, 0, ni)),
    ]
    if has_bias:
        in_specs.append(pl.BlockSpec((m, 1), lambda bi, ni: (0, 0)))
    if f2:
        in_specs.append(pl.BlockSpec((f2, m2), lambda bi, ni: (0, 0)))
        in_specs.append(pl.BlockSpec((1, f2, nb), lambda bi, ni: (bi, 0, ni)))

    return pl.pallas_call(
        body,
        grid=grid,
        in_specs=in_specs,
        out_specs=pl.BlockSpec((1, m, nb), lambda bi, ni: (bi, 0, ni)),
        out_shape=jax.ShapeDtypeStruct((b, m, n), _F32),
    )


@functools.lru_cache(maxsize=None)
def _make_max(r, ncols):
    br = min(r, 256)
    bn = min(ncols, 2048)
    grid = (-(-r // br), -(-ncols // bn))

    def body(a_ref, b_ref, o_ref):
        o_ref[...] = jnp.maximum(a_ref[...], b_ref[...])

    spec = pl.BlockSpec((br, bn), lambda i, j: (i, j))
    return pl.pallas_call(
        body,
        grid=grid,
        in_specs=[spec, spec],
        out_specs=spec,
        out_shape=jax.ShapeDtypeStruct((r, ncols), _F32),
    )


def _cheb_T(xbc, graph):
    ew, n = graph
    b, cin, _ = xbc.shape
    c = b * cin
    x2 = xbc.reshape(c, n)
    t1, t2 = _make_t1t2(n, c)(x2, ew)
    return jnp.concatenate(
        [xbc, t1.reshape(b, cin, n), t2.reshape(b, cin, n)], axis=1)


def _conv_k3(xbc, p, graph, act):
    t = _cheb_T(xbc, graph)
    b, f, n = t.shape
    m = p["W"].shape[2]
    wf = p["W"].reshape(f, m)
    bias = p["b"].reshape(m, 1)
    return _make_mm(f, m, n, b, True, 0, act)(wf, t, bias)


def _conv_k3_commuted(xbc, p, graph):
    ew, n = graph
    b, cin, _ = xbc.shape
    cout = p["W"].shape[2]
    w0, w1, w2 = p["W"][0], p["W"][1], p["W"][2]
    wp = jnp.concatenate([w0 - w2, w1, w2], axis=1)
    bias3 = jnp.concatenate(
        [p["b"], jnp.zeros((2 * cout,), _F32)]).reshape(3 * cout, 1)
    y = _make_mm(cin, 3 * cout, n, b, True, 0, "none")(wp, xbc, bias3)
    c = b * cout
    base = y[:, :cout].reshape(c, n)
    z1 = y[:, cout:2 * cout].reshape(c, n)
    z2 = y[:, 2 * cout:].reshape(c, n)
    out = _make_dec_apply(n, c)(z2, z1, base, ew)
    return out.reshape(b, cout, n)


def _res_block(xbc, p, graph):
    if p["conv1"]["W"].shape[2] <= xbc.shape[1]:
        h = _conv_k3_commuted(xbc, p["conv1"], graph)
    else:
        h = _conv_k3(xbc, p["conv1"], graph, "relu")
    ew, n = graph
    b, c, _ = h.shape
    cin = xbc.shape[1]
    w0, w1, w2 = (p["conv2"]["W"][k] for k in range(3))
    wp = jnp.concatenate([w0 - w2, w1, w2], axis=1)
    bias3 = jnp.concatenate(
        [p["conv2"]["b"], jnp.zeros((2 * c,), _F32)]).reshape(3 * c, 1)
    wsc = p["sc"]["W"][0]
    y = _make_mm(c, 3 * c, n, b, True, cin, "none", m2=c)(
        wp, h, bias3, wsc, xbc)
    cc = b * c
    base = y[:, :c].reshape(cc, n)
    z1 = y[:, c:2 * c].reshape(cc, n)
    z2 = y[:, 2 * c:].reshape(cc, n)
    out = _make_dec_apply(n, cc)(z2, z1, base, ew)
    return out.reshape(b, c, n)


def _pool(t):
    b, c, n = t.shape
    a = t[:, :, 0::2].reshape(b * c, n // 2)
    bb = t[:, :, 1::2].reshape(b * c, n // 2)
    return _make_max(b * c, n // 2)(a, bb).reshape(b, c, n // 2)


def _unpool(t):
    return jnp.repeat(t, 2, axis=2)


def kernel(x, params, edge_src, edge_dst, edge_w):
    graphs = {}
    for i, lvl in enumerate(_LVLS):
        n = _NS[i]
        src = edge_src[lvl].astype(_I32)
        dst = edge_dst[lvl].astype(_I32)
        pk = jnp.bitwise_or(src, dst << 16)
        ew = edge_w[lvl].astype(_F32)
        ew_i = lax.bitcast_convert_type(ew, _I32)
        ewp = jnp.stack(
            [pk.reshape(-1, 16), ew_i.reshape(-1, 16)], axis=1).reshape(-1)
        parts = _make_deg(n)(ewp)
        isd = _make_isd(_round_up(n, 256))(parts)
        wn = _make_wn(n)(ewp, isd)
        wn_i = lax.bitcast_convert_type(-wn, _I32)
        epk = jnp.stack(
            [pk.reshape(-1, 16), wn_i.reshape(-1, 16)], axis=1).reshape(-1)
        graphs[lvl] = (epk, n)

    h = _conv_k3(x, params["enc_conv"], graphs["l5"], "relu")
    e5 = _res_block(h, params["enc_b5"], graphs["l5"])
    e4 = _res_block(_pool(e5), params["enc_b4"], graphs["l4"])
    e3 = _res_block(_pool(e4), params["enc_b3"], graphs["l3"])
    e2 = _res_block(_pool(e3), params["enc_b2"], graphs["l2"])
    e1 = _res_block(_pool(e2), params["enc_b1"], graphs["l1"])
    e0 = _res_block(_pool(e1), params["enc_b0"], graphs["l0"])
    d1 = _res_block(jnp.concatenate([_unpool(e0), e1], axis=1),
                    params["dec_b1"], graphs["l1"])
    d2 = _res_block(jnp.concatenate([_unpool(d1), e2], axis=1),
                    params["dec_b2"], graphs["l2"])
    d3 = _res_block(jnp.concatenate([_unpool(d2), e3], axis=1),
                    params["dec_b3"], graphs["l3"])
    d4 = _res_block(jnp.concatenate([_unpool(d3), e4], axis=1),
                    params["dec_b4"], graphs["l4"])
    d5 = _res_block(jnp.concatenate([_unpool(d4), e5], axis=1),
                    params["dec_b5"], graphs["l5"])

    b, cin, n = d5.shape
    wdec = params["dec_conv"]["W"].reshape(cin, 10)
    return _make_mm(cin, 10, n, b, False, 0, "lsm")(wdec, d5)


# speedup vs baseline: 1.0468x; 1.0209x over previous
import functools
import math

import jax
import jax.numpy as jnp
from jax import lax
from jax.experimental import pallas as pl
from jax.experimental.pallas import tpu as pltpu
from jax.experimental.pallas import tpu_sc as plsc

_KS = 3
_NS = [800, 1600, 3200, 6400, 12800, 25600]
_LVLS = ["l0", "l1", "l2", "l3", "l4", "l5"]
_NW = 32
_F32 = jnp.float32
_I32 = jnp.int32


def _mesh():
    return plsc.VectorSubcoreMesh(core_axis_name="c", subcore_axis_name="s")


_SC_PARAMS = pltpu.CompilerParams(
    needs_layout_passes=False, use_tc_tiling_on_sc=False)


def _wid():
    return lax.axis_index("s") * 2 + lax.axis_index("c")


def _round_up(x, m):
    return (x + m - 1) // m * m


@functools.lru_cache(maxsize=None)
def _make_deg(n):
    e = n * 16
    npad = _round_up(n, 256)
    epw = e // _NW
    ce = min(epw, 1600)
    nch = epw // ce

    @functools.partial(
        pl.kernel,
        out_type=jax.ShapeDtypeStruct((_NW, npad), _F32),
        mesh=_mesh(),
        compiler_params=_SC_PARAMS,
        scratch_types=[
            pltpu.VMEM((npad,), _F32),
            pltpu.VMEM((2 * ce,), _I32),
            pltpu.VMEM((2 * ce,), _I32),
            pltpu.SemaphoreType.DMA,
            pltpu.SemaphoreType.DMA,
        ],
    )
    def deg_kernel(ew_hbm, out_hbm, part_v, e0_v, e1_v, sem0, sem1):
        w = _wid()
        zero16 = jnp.zeros((16,), _F32)

        @plsc.parallel_loop(0, npad // 16, unroll=8)
        def _(i):
            part_v[pl.ds(i * 16, 16)] = zero16
        ebase = 2 * w * epw

        def compute(buf):
            @plsc.parallel_loop(0, ce // 16, unroll=8)
            def _(i):
                pk16 = buf[pl.ds(i * 32, 16)]
                d16 = lax.shift_right_logical(pk16, 16)
                w16 = plsc.bitcast(buf[pl.ds(i * 32 + 16, 16)], _F32)
                plsc.addupdate_scatter(part_v, [d16], w16)

        if nch == 1:
            pltpu.sync_copy(ew_hbm.at[pl.ds(ebase, 2 * ce)], e0_v)
            compute(e0_v)
        else:
            pltpu.async_copy(ew_hbm.at[pl.ds(ebase, 2 * ce)], e0_v, sem0)

            def pair(p, _):
                c0 = 2 * p
                pltpu.async_copy(
                    ew_hbm.at[pl.ds(ebase + (c0 + 1) * 2 * ce, 2 * ce)],
                    e1_v, sem1)
                pltpu.make_async_copy(
                    ew_hbm.at[pl.ds(ebase + c0 * 2 * ce, 2 * ce)],
                    e0_v, sem0).wait()
                compute(e0_v)

                @pl.when(c0 + 2 < nch)
                def _():
                    pltpu.async_copy(
                        ew_hbm.at[pl.ds(ebase + (c0 + 2) * 2 * ce, 2 * ce)],
                        e0_v, sem0)

                pltpu.make_async_copy(
                    ew_hbm.at[pl.ds(ebase + (c0 + 1) * 2 * ce, 2 * ce)],
                    e1_v, sem1).wait()
                compute(e1_v)
                return 0

            lax.fori_loop(0, nch // 2, pair, 0)
        pltpu.sync_copy(part_v, out_hbm.at[w])

    return deg_kernel


@functools.lru_cache(maxsize=None)
def _make_isd(npad):
    def body(parts_ref, out_ref):
        deg = jnp.sum(parts_ref[...], axis=0, keepdims=True) + 1e-6
        out_ref[...] = lax.rsqrt(deg)

    return pl.pallas_call(
        body,
        out_shape=jax.ShapeDtypeStruct((1, npad), _F32),
    )


@functools.lru_cache(maxsize=None)
def _make_wn(n):
    e = n * 16
    npad = _round_up(n, 256)
    epw = e // _NW
    ce = min(epw, 1600)
    nch = epw // ce

    @functools.partial(
        pl.kernel,
        out_type=jax.ShapeDtypeStruct((e,), _F32),
        mesh=_mesh(),
        compiler_params=_SC_PARAMS,
        scratch_types=[
            pltpu.VMEM((npad,), _F32),
            pltpu.VMEM((2 * ce,), _I32),
            pltpu.VMEM((2 * ce,), _I32),
            pltpu.VMEM((ce,), _F32),
            pltpu.SemaphoreType.DMA,
            pltpu.SemaphoreType.DMA,
        ],
    )
    def wn_kernel(ew_hbm, isd_hbm, out_hbm, isd_v, e0_v, e1_v, o_v,
                  sem0, sem1):
        w = _wid()
        pltpu.sync_copy(isd_hbm.at[0], isd_v)
        ebase = w * epw

        def compute(buf, ch):
            @plsc.parallel_loop(0, ce // 16, unroll=8)
            def _(i):
                pk16 = buf[pl.ds(i * 32, 16)]
                s16 = jnp.bitwise_and(pk16, 0xFFFF)
                d16 = lax.shift_right_logical(pk16, 16)
                w16 = plsc.bitcast(buf[pl.ds(i * 32 + 16, 16)], _F32)
                a = plsc.load_gather(isd_v, [s16])
                b = plsc.load_gather(isd_v, [d16])
                o_v[pl.ds(i * 16, 16)] = w16 * a * b
            pltpu.sync_copy(o_v, out_hbm.at[pl.ds(ebase + ch * ce, ce)])

        if nch == 1:
            pltpu.sync_copy(ew_hbm.at[pl.ds(2 * ebase, 2 * ce)], e0_v)
            compute(e0_v, 0)
        else:
            pltpu.async_copy(ew_hbm.at[pl.ds(2 * ebase, 2 * ce)], e0_v, sem0)

            def pair(p, _):
                c0 = 2 * p
                pltpu.async_copy(
                    ew_hbm.at[pl.ds(2 * (ebase + (c0 + 1) * ce), 2 * ce)],
                    e1_v, sem1)
                pltpu.make_async_copy(
                    ew_hbm.at[pl.ds(2 * (ebase + c0 * ce), 2 * ce)],
                    e0_v, sem0).wait()
                compute(e0_v, c0)

                @pl.when(c0 + 2 < nch)
                def _():
                    pltpu.async_copy(
                        ew_hbm.at[pl.ds(2 * (ebase + (c0 + 2) * ce), 2 * ce)],
                        e0_v, sem0)

                pltpu.make_async_copy(
                    ew_hbm.at[pl.ds(2 * (ebase + (c0 + 1) * ce), 2 * ce)],
                    e1_v, sem1).wait()
                compute(e1_v, c0 + 1)
                return 0

            lax.fori_loop(0, nch // 2, pair, 0)

    return wn_kernel


@functools.lru_cache(maxsize=None)
def _make_apply(n, c_cols, cheb):
    e = n * 16
    ce = 3200
    nch = e // ce
    half = nch // 2
    budget = 112000
    ncol = max(1, min(budget // (2 * n), 32, -(-c_cols // _NW)))
    nsweep = -(-c_cols // (_NW * ncol))

    scratch = [
        pltpu.VMEM((ncol * n,), _F32),
        pltpu.VMEM((ncol * n,), _F32),
        pltpu.VMEM((2 * ce,), _I32),
        pltpu.VMEM((2 * ce,), _I32),
        pltpu.SemaphoreType.DMA,
        pltpu.SemaphoreType.DMA,
    ]

    def body(x_hbm, ew_hbm, *rest):
        if cheb:
            prev_hbm, out_hbm, x_v, a_v, e0_v, e1_v, sem0, sem1 = rest
        else:
            out_hbm, x_v, a_v, e0_v, e1_v, sem0, sem1 = rest
        w = _wid()

        unroll = max(1, min(16, 64 // max(ncol, 1)))

        def compute(buf):
            @plsc.parallel_loop(0, ce // 16, unroll=unroll)
            def _(i):
                pk16 = buf[pl.ds(i * 32, 16)]
                s16 = jnp.bitwise_and(pk16, 0xFFFF)
                d16 = lax.shift_right_logical(pk16, 16)
                w16 = plsc.bitcast(buf[pl.ds(i * 32 + 16, 16)], _F32)
                for j in range(ncol):
                    v = plsc.load_gather(x_v, [s16 + j * n])
                    plsc.addupdate_scatter(a_v, [d16 + j * n], v * w16)

        for sw in range(nsweep):
            base = (sw * _NW + w) * ncol

            for j in range(ncol):
                col = base + j

                @pl.when(col < c_cols)
                def _():
                    pltpu.sync_copy(x_hbm.at[col], x_v.at[pl.ds(j * n, n)])
                    pltpu.sync_copy(x_hbm.at[col], a_v.at[pl.ds(j * n, n)])

            @pl.when(base < c_cols)
            def _():
                pltpu.async_copy(ew_hbm.at[pl.ds(0, 2 * ce)], e0_v, sem0)

                def pair(p, _):
                    c0 = 2 * p
                    pltpu.async_copy(
                        ew_hbm.at[pl.ds((c0 + 1) * 2 * ce, 2 * ce)], e1_v, sem1)
                    pltpu.make_async_copy(
                        ew_hbm.at[pl.ds(c0 * 2 * ce, 2 * ce)], e0_v, sem0).wait()
                    compute(e0_v)

                    @pl.when(c0 + 2 < nch)
                    def _():
                        pltpu.async_copy(
                            ew_hbm.at[pl.ds((c0 + 2) * 2 * ce, 2 * ce)],
                            e0_v, sem0)

                    pltpu.make_async_copy(
                        ew_hbm.at[pl.ds((c0 + 1) * 2 * ce, 2 * ce)],
                        e1_v, sem1).wait()
                    compute(e1_v)
                    return 0

                lax.fori_loop(0, half, pair, 0)

            for j in range(ncol):
                col = base + j

                @pl.when(col < c_cols)
                def _():
                    if not cheb:
                        pltpu.sync_copy(a_v.at[pl.ds(j * n, n)], out_hbm.at[col])
                    else:
                        pltpu.sync_copy(prev_hbm.at[col], x_v.at[pl.ds(j * n, n)])

                        jj = j * n

                        @plsc.parallel_loop(0, n // 16, unroll=8)
                        def _(i):
                            av = a_v[pl.ds(jj + i * 16, 16)]
                            pv = x_v[pl.ds(jj + i * 16, 16)]
                            x_v[pl.ds(jj + i * 16, 16)] = 2.0 * av - pv
                        pltpu.sync_copy(x_v.at[pl.ds(j * n, n)], out_hbm.at[col])

    return functools.partial(
        pl.kernel,
        out_type=jax.ShapeDtypeStruct((c_cols, n), _F32),
        mesh=_mesh(),
        compiler_params=_SC_PARAMS,
        scratch_types=scratch,
    )(body)


@functools.lru_cache(maxsize=None)
def _make_dec_apply(n, c_cols):
    e = n * 16
    ce = 3200
    nch = e // ce
    half = nch // 2
    budget = 112000
    ncol = max(1, min(budget // (2 * n), 32, -(-c_cols // _NW)))
    nsweep = -(-c_cols // (_NW * ncol))

    scratch = [
        pltpu.VMEM((ncol * n,), _F32),
        pltpu.VMEM((ncol * n,), _F32),
        pltpu.VMEM((2 * ce,), _I32),
        pltpu.VMEM((2 * ce,), _I32),
        pltpu.SemaphoreType.DMA,
        pltpu.SemaphoreType.DMA,
    ]

    @functools.partial(
        pl.kernel,
        out_type=jax.ShapeDtypeStruct((c_cols, n), _F32),
        mesh=_mesh(),
        compiler_params=_SC_PARAMS,
        scratch_types=scratch,
    )
    def dec_kernel(z2_hbm, z1_hbm, base_hbm, ew_hbm, out_hbm,
                   x_v, a_v, e0_v, e1_v, sem0, sem1):
        w = _wid()
        unroll = max(1, min(16, 64 // max(ncol, 1)))

        def compute(buf):
            @plsc.parallel_loop(0, ce // 16, unroll=unroll)
            def _(i):
                pk16 = buf[pl.ds(i * 32, 16)]
                s16 = jnp.bitwise_and(pk16, 0xFFFF)
                d16 = lax.shift_right_logical(pk16, 16)
                w16 = plsc.bitcast(buf[pl.ds(i * 32 + 16, 16)], _F32)
                for j in range(ncol):
                    v = plsc.load_gather(x_v, [s16 + j * n])
                    plsc.addupdate_scatter(a_v, [d16 + j * n], v * w16)

        def edge_pass():
            pltpu.async_copy(ew_hbm.at[pl.ds(0, 2 * ce)], e0_v, sem0)

            def pair(p, _):
                c0 = 2 * p
                pltpu.async_copy(
                    ew_hbm.at[pl.ds((c0 + 1) * 2 * ce, 2 * ce)], e1_v, sem1)
                pltpu.make_async_copy(
                    ew_hbm.at[pl.ds(c0 * 2 * ce, 2 * ce)], e0_v, sem0).wait()
                compute(e0_v)

                @pl.when(c0 + 2 < nch)
                def _():
                    pltpu.async_copy(
                        ew_hbm.at[pl.ds((c0 + 2) * 2 * ce, 2 * ce)],
                        e0_v, sem0)

                pltpu.make_async_copy(
                    ew_hbm.at[pl.ds((c0 + 1) * 2 * ce, 2 * ce)],
                    e1_v, sem1).wait()
                compute(e1_v)
                return 0

            lax.fori_loop(0, half, pair, 0)

        for sw in range(nsweep):
            base = (sw * _NW + w) * ncol

            for j in range(ncol):
                col = base + j

                @pl.when(col < c_cols)
                def _():
                    pltpu.sync_copy(z2_hbm.at[col], x_v.at[pl.ds(j * n, n)])
                    pltpu.sync_copy(z2_hbm.at[col], a_v.at[pl.ds(j * n, n)])

            @pl.when(base < c_cols)
            def _():
                edge_pass()

            for j in range(ncol):
                col = base + j

                @pl.when(col < c_cols)
                def _():
                    pltpu.sync_copy(z1_hbm.at[col], x_v.at[pl.ds(j * n, n)])

            @pl.when(base < c_cols)
            def _():
                @plsc.parallel_loop(0, ncol * n // 16, unroll=8)
                def _(i):
                    t = x_v[pl.ds(i * 16, 16)] + 2.0 * a_v[pl.ds(i * 16, 16)]
                    x_v[pl.ds(i * 16, 16)] = t
                    a_v[pl.ds(i * 16, 16)] = t

                edge_pass()

            for j in range(ncol):
                col = base + j
                jj = j * n

                @pl.when(col < c_cols)
                def _():
                    pltpu.sync_copy(base_hbm.at[col], x_v.at[pl.ds(jj, n)])

                    @plsc.parallel_loop(0, n // 16, unroll=8)
                    def _(i):
                        t = x_v[pl.ds(jj + i * 16, 16)] + a_v[pl.ds(jj + i * 16, 16)]
                        x_v[pl.ds(jj + i * 16, 16)] = jnp.maximum(t, 0.0)

                    pltpu.sync_copy(x_v.at[pl.ds(jj, n)], out_hbm.at[col])

    return dec_kernel


@functools.lru_cache(maxsize=None)
def _make_t1t2(n, c_cols):
    e = n * 16
    ce = 3200
    nch = e // ce
    half = nch // 2
    budget = 112000
    ncol = max(1, min(budget // (3 * n), 32, -(-c_cols // _NW)))
    nsweep = -(-c_cols // (_NW * ncol))

    scratch = [
        pltpu.VMEM((ncol * n,), _F32),
        pltpu.VMEM((ncol * n,), _F32),
        pltpu.VMEM((ncol * n,), _F32),
        pltpu.VMEM((2 * ce,), _I32),
        pltpu.VMEM((2 * ce,), _I32),
        pltpu.SemaphoreType.DMA,
        pltpu.SemaphoreType.DMA,
    ]

    @functools.partial(
        pl.kernel,
        out_type=[jax.ShapeDtypeStruct((c_cols, n), _F32),
                  jax.ShapeDtypeStruct((c_cols, n), _F32)],
        mesh=_mesh(),
        compiler_params=_SC_PARAMS,
        scratch_types=scratch,
    )
    def t1t2_kernel(x_hbm, ew_hbm, t1_hbm, t2_hbm,
                    x_v, t_v, a_v, e0_v, e1_v, sem0, sem1):
        w = _wid()
        unroll = max(1, min(16, 64 // max(ncol, 1)))

        def make_compute(src_v):
            def compute(buf):
                @plsc.parallel_loop(0, ce // 16, unroll=unroll)
                def _(i):
                    pk16 = buf[pl.ds(i * 32, 16)]
                    s16 = jnp.bitwise_and(pk16, 0xFFFF)
                    d16 = lax.shift_right_logical(pk16, 16)
                    w16 = plsc.bitcast(buf[pl.ds(i * 32 + 16, 16)], _F32)
                    for j in range(ncol):
                        v = plsc.load_gather(src_v, [s16 + j * n])
                        plsc.addupdate_scatter(a_v, [d16 + j * n], v * w16)
            return compute

        def edge_pass(compute):
            pltpu.async_copy(ew_hbm.at[pl.ds(0, 2 * ce)], e0_v, sem0)

            def pair(p, _):
                c0 = 2 * p
                pltpu.async_copy(
                    ew_hbm.at[pl.ds((c0 + 1) * 2 * ce, 2 * ce)], e1_v, sem1)
                pltpu.make_async_copy(
                    ew_hbm.at[pl.ds(c0 * 2 * ce, 2 * ce)], e0_v, sem0).wait()
                compute(e0_v)

                @pl.when(c0 + 2 < nch)
                def _():
                    pltpu.async_copy(
                        ew_hbm.at[pl.ds((c0 + 2) * 2 * ce, 2 * ce)],
                        e0_v, sem0)

                pltpu.make_async_copy(
                    ew_hbm.at[pl.ds((c0 + 1) * 2 * ce, 2 * ce)],
                    e1_v, sem1).wait()
                compute(e1_v)
                return 0

            lax.fori_loop(0, half, pair, 0)

        for sw in range(nsweep):
            base = (sw * _NW + w) * ncol

            for j in range(ncol):
                col = base + j

                @pl.when(col < c_cols)
                def _():
                    pltpu.sync_copy(x_hbm.at[col], x_v.at[pl.ds(j * n, n)])
                    pltpu.sync_copy(x_hbm.at[col], a_v.at[pl.ds(j * n, n)])

            @pl.when(base < c_cols)
            def _():
                edge_pass(make_compute(x_v))

            for j in range(ncol):
                col = base + j

                @pl.when(col < c_cols)
                def _():
                    pltpu.sync_copy(a_v.at[pl.ds(j * n, n)], t1_hbm.at[col])

            @pl.when(base < c_cols)
            def _():
                @plsc.parallel_loop(0, ncol * n // 16, unroll=8)
                def _(i):
                    t2x = 2.0 * a_v[pl.ds(i * 16, 16)]
                    t_v[pl.ds(i * 16, 16)] = t2x
                    a_v[pl.ds(i * 16, 16)] = t2x

                edge_pass(make_compute(t_v))

            for j in range(ncol):
                col = base + j
                jj = j * n

                @pl.when(col < c_cols)
                def _():
                    @plsc.parallel_loop(0, n // 16, unroll=8)
                    def _(i):
                        t = a_v[pl.ds(jj + i * 16, 16)] - x_v[pl.ds(jj + i * 16, 16)]
                        x_v[pl.ds(jj + i * 16, 16)] = t

                    pltpu.sync_copy(x_v.at[pl.ds(jj, n)], t2_hbm.at[col])

    return t1t2_kernel

@functools.lru_cache(maxsize=None)
def _make_mm(f, m, n, b, has_bias, f2, act, m2=None):
    m2 = m if m2 is None else m2
    nb = min(1024, n)
    grid = (b, -(-n // nb))

    def body(*refs):
        idx = 0
        w_ref = refs[idx]; idx += 1
        t_ref = refs[idx]; idx += 1
        if has_bias:
            bias_ref = refs[idx]; idx += 1
        if f2:
            w2_ref = refs[idx]; idx += 1
            t2_ref = refs[idx]; idx += 1
        out_ref = refs[idx]
        y = lax.dot_general(
            w_ref[...], t_ref[0],
            (((0,), (0,)), ((), ())),
            precision=lax.Precision.HIGHEST,
            preferred_element_type=_F32,
        )
        if f2:
            y2 = lax.dot_general(
                w2_ref[...], t2_ref[0],
                (((0,), (0,)), ((), ())),
                precision=lax.Precision.HIGHEST,
                preferred_element_type=_F32,
            )
            if m2 == m:
                y = y + y2
            else:
                y = jnp.concatenate([y[:m2] + y2, y[m2:]], axis=0)
        if has_bias:
            y = y + bias_ref[...]
        if act == "relu":
            y = jnp.maximum(y, 0.0)
        elif act == "lsm":
            y = y - jnp.max(y, axis=0, keepdims=True)
            y = y - jnp.log(jnp.sum(jnp.exp(y), axis=0, keepdims=True))
        out_ref[0] = y

    in_specs = [
        pl.BlockSpec((f, m), lambda bi, ni: (0, 0)),
        pl.BlockSpec((1, f, nb), lambda bi, ni: (bi, 0, ni)),
    ]
    if has_bias:
        in_specs.append(pl.BlockSpec((m, 1), lambda bi, ni: (0, 0)))
    if f2:
        in_specs.append(pl.BlockSpec((f2, m2), lambda bi, ni: (0, 0)))
        in_specs.append(pl.BlockSpec((1, f2, nb), lambda bi, ni: (bi, 0, ni)))

    return pl.pallas_call(
        body,
        grid=grid,
        in_specs=in_specs,
        out_specs=pl.BlockSpec((1, m, nb), lambda bi, ni: (bi, 0, ni)),
        out_shape=jax.ShapeDtypeStruct((b, m, n), _F32),
    )


@functools.lru_cache(maxsize=None)
def _make_mm3(f, m3, n, b, f2):
    nb = min(1024, n)
    grid = (b, -(-n // nb))
    m = 3 * m3

    def body(*refs):
        idx = 0
        w_ref = refs[idx]; idx += 1
        t_ref = refs[idx]; idx += 1
        bias_ref = refs[idx]; idx += 1
        if f2:
            w2_ref = refs[idx]; idx += 1
            t2_ref = refs[idx]; idx += 1
        o_base, o_z1, o_z2 = refs[idx], refs[idx + 1], refs[idx + 2]
        y = lax.dot_general(
            w_ref[...], t_ref[0],
            (((0,), (0,)), ((), ())),
            precision=lax.Precision.HIGHEST,
            preferred_element_type=_F32,
        )
        base = y[:m3] + bias_ref[...]
        if f2:
            base = base + lax.dot_general(
                w2_ref[...], t2_ref[0],
                (((0,), (0,)), ((), ())),
                precision=lax.Precision.HIGHEST,
                preferred_element_type=_F32,
            )
        o_base[...] = base
        o_z1[...] = y[m3:2 * m3]
        o_z2[...] = y[2 * m3:]

    in_specs = [
        pl.BlockSpec((f, m), lambda bi, ni: (0, 0)),
        pl.BlockSpec((1, f, nb), lambda bi, ni: (bi, 0, ni)),
        pl.BlockSpec((m3, 1), lambda bi, ni: (0, 0)),
    ]
    if f2:
        in_specs.append(pl.BlockSpec((f2, m3), lambda bi, ni: (0, 0)))
        in_specs.append(pl.BlockSpec((1, f2, nb), lambda bi, ni: (bi, 0, ni)))
    ospec = pl.BlockSpec((m3, nb), lambda bi, ni: (bi, ni))
    oshape = jax.ShapeDtypeStruct((b * m3, n), _F32)

    return pl.pallas_call(
        body,
        grid=grid,
        in_specs=in_specs,
        out_specs=[ospec, ospec, ospec],
        out_shape=[oshape, oshape, oshape],
    )


@functools.lru_cache(maxsize=None)
def _make_mmk3(cin, m, n, b, act):
    nb = min(1024, n)
    grid = (b, -(-n // nb))

    def body(w_ref, x_ref, t1_ref, t2_ref, bias_ref, out_ref):
        kw = dict(precision=lax.Precision.HIGHEST,
                  preferred_element_type=_F32)
        dn = (((0,), (0,)), ((), ()))
        y = lax.dot_general(w_ref[:cin], x_ref[0], dn, **kw)
        y = y + lax.dot_general(w_ref[cin:2 * cin], t1_ref[...], dn, **kw)
        y = y + lax.dot_general(w_ref[2 * cin:], t2_ref[...], dn, **kw)
        y = y + bias_ref[...]
        if act == "relu":
            y = jnp.maximum(y, 0.0)
        out_ref[0] = y

    in_specs = [
        pl.BlockSpec((3 * cin, m), lambda bi, ni: (0, 0)),
        pl.BlockSpec((1, cin, nb), lambda bi, ni: (bi, 0, ni)),
        pl.BlockSpec((cin, nb), lambda bi, ni: (bi, ni)),
        pl.BlockSpec((cin, nb), lambda bi, ni: (bi, ni)),
        pl.BlockSpec((m, 1), lambda bi, ni: (0, 0)),
    ]

    return pl.pallas_call(
        body,
        grid=grid,
        in_specs=in_specs,
        out_specs=pl.BlockSpec((1, m, nb), lambda bi, ni: (bi, 0, ni)),
        out_shape=jax.ShapeDtypeStruct((b, m, n), _F32),
    )

@functools.lru_cache(maxsize=None)
def _make_max(r, ncols):
    br = min(r, 256)
    bn = min(ncols, 2048)
    grid = (-(-r // br), -(-ncols // bn))

    def body(a_ref, b_ref, o_ref):
        o_ref[...] = jnp.maximum(a_ref[...], b_ref[...])

    spec = pl.BlockSpec((br, bn), lambda i, j: (i, j))
    return pl.pallas_call(
        body,
        grid=grid,
        in_specs=[spec, spec],
        out_specs=spec,
        out_shape=jax.ShapeDtypeStruct((r, ncols), _F32),
    )


def _cheb_T(xbc, graph):
    ew, n = graph
    b, cin, _ = xbc.shape
    c = b * cin
    x2 = xbc.reshape(c, n)
    t1, t2 = _make_t1t2(n, c)(x2, ew)
    return jnp.concatenate(
        [xbc, t1.reshape(b, cin, n), t2.reshape(b, cin, n)], axis=1)


def _conv_k3(xbc, p, graph, act):
    ew, n = graph
    b, cin, _ = xbc.shape
    c = b * cin
    t1, t2 = _make_t1t2(n, c)(xbc.reshape(c, n), ew)
    m = p["W"].shape[2]
    wf = p["W"].reshape(3 * cin, m)
    bias = p["b"].reshape(m, 1)
    if cin % 8 == 0:
        return _make_mmk3(cin, m, n, b, act)(wf, xbc, t1, t2, bias)
    t = jnp.concatenate(
        [xbc, t1.reshape(b, cin, n), t2.reshape(b, cin, n)], axis=1)
    return _make_mm(3 * cin, m, n, b, True, 0, act)(wf, t, bias)


def _conv_k3_commuted(xbc, p, graph):
    ew, n = graph
    b, cin, _ = xbc.shape
    cout = p["W"].shape[2]
    w0, w1, w2 = p["W"][0], p["W"][1], p["W"][2]
    wp = jnp.concatenate([w0 - w2, w1, w2], axis=1)
    bias = p["b"].reshape(cout, 1)
    base, z1, z2 = _make_mm3(cin, cout, n, b, 0)(wp, xbc, bias)
    c = b * cout
    out = _make_dec_apply(n, c)(z2, z1, base, ew)
    return out.reshape(b, cout, n)


def _res_block(xbc, p, graph):
    if p["conv1"]["W"].shape[2] <= xbc.shape[1]:
        h = _conv_k3_commuted(xbc, p["conv1"], graph)
    else:
        h = _conv_k3(xbc, p["conv1"], graph, "relu")
    ew, n = graph
    b, c, _ = h.shape
    cin = xbc.shape[1]
    w0, w1, w2 = (p["conv2"]["W"][k] for k in range(3))
    wp = jnp.concatenate([w0 - w2, w1, w2], axis=1)
    bias = p["conv2"]["b"].reshape(c, 1)
    wsc = p["sc"]["W"][0]
    base, z1, z2 = _make_mm3(c, c, n, b, cin)(wp, h, bias, wsc, xbc)
    cc = b * c
    out = _make_dec_apply(n, cc)(z2, z1, base, ew)
    return out.reshape(b, c, n)


def _pool(t):
    b, c, n = t.shape
    a = t[:, :, 0::2].reshape(b * c, n // 2)
    bb = t[:, :, 1::2].reshape(b * c, n // 2)
    return _make_max(b * c, n // 2)(a, bb).reshape(b, c, n // 2)


def _unpool(t):
    return jnp.repeat(t, 2, axis=2)


def kernel(x, params, edge_src, edge_dst, edge_w):
    graphs = {}
    for i, lvl in enumerate(_LVLS):
        n = _NS[i]
        src = edge_src[lvl].astype(_I32)
        dst = edge_dst[lvl].astype(_I32)
        pk = jnp.bitwise_or(src, dst << 16)
        ew = edge_w[lvl].astype(_F32)
        ew_i = lax.bitcast_convert_type(ew, _I32)
        ewp = jnp.stack(
            [pk.reshape(-1, 16), ew_i.reshape(-1, 16)], axis=1).reshape(-1)
        parts = _make_deg(n)(ewp)
        isd = _make_isd(_round_up(n, 256))(parts)
        wn = _make_wn(n)(ewp, isd)
        wn_i = lax.bitcast_convert_type(-wn, _I32)
        epk = jnp.stack(
            [pk.reshape(-1, 16), wn_i.reshape(-1, 16)], axis=1).reshape(-1)
        graphs[lvl] = (epk, n)

    h = _conv_k3(x, params["enc_conv"], graphs["l5"], "relu")
    e5 = _res_block(h, params["enc_b5"], graphs["l5"])
    e4 = _res_block(_pool(e5), params["enc_b4"], graphs["l4"])
    e3 = _res_block(_pool(e4), params["enc_b3"], graphs["l3"])
    e2 = _res_block(_pool(e3), params["enc_b2"], graphs["l2"])
    e1 = _res_block(_pool(e2), params["enc_b1"], graphs["l1"])
    e0 = _res_block(_pool(e1), params["enc_b0"], graphs["l0"])
    d1 = _res_block(jnp.concatenate([_unpool(e0), e1], axis=1),
                    params["dec_b1"], graphs["l1"])
    d2 = _res_block(jnp.concatenate([_unpool(d1), e2], axis=1),
                    params["dec_b2"], graphs["l2"])
    d3 = _res_block(jnp.concatenate([_unpool(d2), e3], axis=1),
                    params["dec_b3"], graphs["l3"])
    d4 = _res_block(jnp.concatenate([_unpool(d3), e4], axis=1),
                    params["dec_b4"], graphs["l4"])
    d5 = _res_block(jnp.concatenate([_unpool(d4), e5], axis=1),
                    params["dec_b5"], graphs["l5"])

    b, cin, n = d5.shape
    wdec = params["dec_conv"]["W"].reshape(cin, 10)
    return _make_mm(cin, 10, n, b, False, 0, "lsm")(wdec, d5)
